# Initial kernel scaffold; baseline (speedup 1.0000x reference)
import jax
import jax.numpy as jnp
from jax.experimental import pallas as pl


def kernel(x):
    raise NotImplementedError("write your pallas kernel here")


# speedup vs baseline: 181.5994x; 181.5994x over previous
import functools

import jax
import jax.numpy as jnp
from jax.experimental import pallas as pl

_K = 256
_ROWS_PER_BLOCK = 16


def _topk_mask_block(x_ref, o_ref, *, k):
    xb = x_ref[...]
    u = jax.lax.bitcast_convert_type(xb, jnp.int32) & jnp.int32(0x7FFFFFFF)
    r = xb.shape[0]
    lo = jnp.zeros((r, 1), jnp.int32)
    hi = jnp.full((r, 1), 0x7F800000, jnp.int32)
    for _ in range(31):
        mid = lo + ((hi - lo + 1) >> 1)
        cnt = jnp.sum((u >= mid).astype(jnp.int32), axis=1, keepdims=True)
        pred = cnt >= k
        lo = jnp.where(pred, mid, lo)
        hi = jnp.where(pred, hi, mid - 1)
    o_ref[...] = jnp.where(u >= lo, xb, 0.0)


def kernel(x):
    b, n = x.shape
    r = _ROWS_PER_BLOCK
    grid = (b // r,)
    return pl.pallas_call(
        functools.partial(_topk_mask_block, k=_K),
        grid=grid,
        in_specs=[pl.BlockSpec((r, n), lambda i: (i, 0))],
        out_specs=pl.BlockSpec((r, n), lambda i: (i, 0)),
        out_shape=jax.ShapeDtypeStruct((b, n), x.dtype),
    )(x)
